# baseline (speedup 1.0000x reference)
import jax
import jax.numpy as jnp
from jax.experimental import pallas as pl
from jax.experimental.pallas import tpu as pltpu

_GROUP = 12500
_B = 5000
_LAG = 3
_RING = 4
_EPS = 1e-6


def _gn_kernel(h_ref, w_ref, b_ref, ms_ref, o_ref, slab_ref, sums_ref, coef_ref):
    s = pl.program_id(0)
    n_in = pl.num_programs(0) - _LAG
    inv_n = 1.0 / _GROUP

    @pl.when(s < n_in)
    def _ingest():
        x = h_ref[...]
        slot = jax.lax.rem(s, _RING)
        slab_ref[pl.ds(slot * _B, _B), :] = x
        pos = jax.lax.rem(s * _B, _GROUP)
        seg = jax.lax.div(s * _B, _GROUP)
        split = _GROUP - pos

        def psums(xm):
            return (jnp.sum(xm, axis=0, keepdims=True),
                    jnp.sum(xm * xm, axis=0, keepdims=True))

        @pl.when(pos == 0)
        def _():
            ps, pss = psums(x)
            sums_ref[pl.ds(2 * seg, 1), :] = ps
            sums_ref[pl.ds(2 * seg + 1, 1), :] = pss

        @pl.when((pos > 0) & (pos + _B <= _GROUP))
        def _():
            ps, pss = psums(x)
            sums_ref[pl.ds(2 * seg, 1), :] += ps
            sums_ref[pl.ds(2 * seg + 1, 1), :] += pss

        @pl.when(pos + _B > _GROUP)
        def _():
            rowid = jax.lax.broadcasted_iota(jnp.int32, (_B, 256), 0)
            lo = rowid < split
            ps, pss = psums(jnp.where(lo, x, 0.0))
            sums_ref[pl.ds(2 * seg, 1), :] += ps
            sums_ref[pl.ds(2 * seg + 1, 1), :] += pss
            ps2, pss2 = psums(jnp.where(lo, 0.0, x))
            sums_ref[pl.ds(2 * seg + 2, 1), :] = ps2
            sums_ref[pl.ds(2 * seg + 3, 1), :] = pss2

    @pl.when(s >= _LAG)
    def _emit():
        e = s - _LAG
        pos = jax.lax.rem(e * _B, _GROUP)
        seg = jax.lax.div(e * _B, _GROUP)
        split = _GROUP - pos
        straddle = pos + _B > _GROUP

        def finalize(j):
            sm = sums_ref[pl.ds(2 * j, 1), :]
            ss = sums_ref[pl.ds(2 * j + 1, 1), :]
            m = sm * inv_n
            mm = m * ms_ref[...]
            var = ss * inv_n - (2.0 * m - mm) * mm
            a = w_ref[...] * jax.lax.rsqrt(var + _EPS)
            coef_ref[pl.ds(2 * j, 1), :] = a
            coef_ref[pl.ds(2 * j + 1, 1), :] = b_ref[...] - a * mm

        @pl.when(pos == 0)
        def _():
            finalize(seg)

        @pl.when(straddle)
        def _():
            finalize(seg + 1)

        slot = jax.lax.rem(e, _RING)
        y = slab_ref[pl.ds(slot * _B, _B), :]
        a0 = coef_ref[pl.ds(2 * seg, 1), :]
        c0 = coef_ref[pl.ds(2 * seg + 1, 1), :]

        @pl.when(jnp.logical_not(straddle))
        def _():
            o_ref[...] = y * a0 + c0

        @pl.when(straddle)
        def _():
            rowid = jax.lax.broadcasted_iota(jnp.int32, (_B, 256), 0)
            a1 = coef_ref[pl.ds(2 * seg + 2, 1), :]
            c1 = coef_ref[pl.ds(2 * seg + 3, 1), :]
            o_ref[...] = jnp.where(rowid < split, y * a0 + c0, y * a1 + c1)


def kernel(h, weight, bias, mean_scale):
    n_rows, d = h.shape
    n_blk = n_rows // _B
    hf = h.astype(jnp.float32)
    w2 = weight.astype(jnp.float32).reshape(1, d)
    b2 = bias.astype(jnp.float32).reshape(1, d)
    ms2 = mean_scale.astype(jnp.float32).reshape(1, d)

    out = pl.pallas_call(
        _gn_kernel,
        grid=(n_blk + _LAG,),
        in_specs=[
            pl.BlockSpec((_B, d), lambda s: (jnp.minimum(s, n_blk - 1), 0)),
            pl.BlockSpec((1, d), lambda s: (0, 0)),
            pl.BlockSpec((1, d), lambda s: (0, 0)),
            pl.BlockSpec((1, d), lambda s: (0, 0)),
        ],
        out_specs=pl.BlockSpec(
            (_B, d), lambda s: (jnp.maximum(s - _LAG, 0), 0)
        ),
        out_shape=jax.ShapeDtypeStruct((n_rows, d), jnp.float32),
        scratch_shapes=[
            pltpu.VMEM((_RING * _B, 256), jnp.float32),
            pltpu.VMEM((16, 256), jnp.float32),
            pltpu.VMEM((16, 256), jnp.float32),
        ],
    )(hf, w2, b2, ms2)

    return out.astype(h.dtype)
